# Initial kernel scaffold; baseline (speedup 1.0000x reference)
import jax
import jax.numpy as jnp
from jax.experimental import pallas as pl


def kernel(edge_index, edge_dist, node_emb, eW, eb, mW1, mb1, mW2, mb2, sW, sb, uW1, ub1, uW2, ub2, oW1, ob1, oW2, ob2, rW1, rb1, rW2, rb2):
    raise NotImplementedError("write your pallas kernel here")


# speedup vs baseline: 1.1004x; 1.1004x over previous
import functools

import jax
import jax.numpy as jnp
from jax.experimental import pallas as pl
from jax.experimental.pallas import tpu as pltpu

N = 10000
E = 160000
H = 256
T = 128
DEPTH = 4

BE = 2000
BN = 2000


def _silu(x):
    return x * jax.nn.sigmoid(x)


def _edge_mlp_body(g_ref, de_ref, We_ref, b1_ref, W2_ref, b2_ref, sW_ref,
                   sb_ref, m_ref, deo_ref):
    de = de_ref[...]
    t = g_ref[...] + jnp.dot(de, We_ref[...],
                             preferred_element_type=jnp.float32) + b1_ref[...]
    h = _silu(t)
    msg = _silu(jnp.dot(h, W2_ref[...],
                        preferred_element_type=jnp.float32) + b2_ref[...])
    deo_ref[...] = de + msg
    ew = jax.nn.sigmoid(jnp.dot(msg, sW_ref[...],
                                preferred_element_type=jnp.float32) + sb_ref[...])
    m_ref[...] = msg * ew


def _edge_mlp(g, de, We, b1, W2, b2, sW, sb):
    grid = E // BE
    eb = lambda i: (i, 0)
    wb = lambda i: (0, 0)
    return pl.pallas_call(
        _edge_mlp_body,
        grid=(grid,),
        in_specs=[
            pl.BlockSpec((BE, H), eb),
            pl.BlockSpec((BE, H), eb),
            pl.BlockSpec((H, H), wb),
            pl.BlockSpec((1, H), wb),
            pl.BlockSpec((H, H), wb),
            pl.BlockSpec((1, H), wb),
            pl.BlockSpec((H, 1), wb),
            pl.BlockSpec((1, 1), wb),
        ],
        out_specs=[pl.BlockSpec((BE, H), eb), pl.BlockSpec((BE, H), eb)],
        out_shape=[jax.ShapeDtypeStruct((E, H), jnp.float32),
                   jax.ShapeDtypeStruct((E, H), jnp.float32)],
    )(g, de, We, b1, W2, b2, sW, sb)


def _node_update_body(msum_ref, feat_ref, W1_ref, b1_ref, W2_ref, b2_ref,
                      Ws_ref, Wd_ref, out_ref, p_ref, q_ref):
    feat = feat_ref[...]
    t = _silu(jnp.dot(msum_ref[...] + feat, W1_ref[...],
                      preferred_element_type=jnp.float32) + b1_ref[...])
    nf = feat + jnp.dot(t, W2_ref[...],
                        preferred_element_type=jnp.float32) + b2_ref[...]
    out_ref[...] = nf
    p_ref[...] = jnp.dot(nf, Ws_ref[...], preferred_element_type=jnp.float32)
    q_ref[...] = jnp.dot(nf, Wd_ref[...], preferred_element_type=jnp.float32)


def _node_update(msum, feat, W1, b1, W2, b2, Ws, Wd):
    grid = N // BN
    nb = lambda i: (i, 0)
    wb = lambda i: (0, 0)
    return pl.pallas_call(
        _node_update_body,
        grid=(grid,),
        in_specs=[
            pl.BlockSpec((BN, H), nb),
            pl.BlockSpec((BN, H), nb),
            pl.BlockSpec((H, H), wb),
            pl.BlockSpec((1, H), wb),
            pl.BlockSpec((H, H), wb),
            pl.BlockSpec((1, H), wb),
            pl.BlockSpec((H, H), wb),
            pl.BlockSpec((H, H), wb),
        ],
        out_specs=[pl.BlockSpec((BN, H), nb), pl.BlockSpec((BN, H), nb),
                   pl.BlockSpec((BN, H), nb)],
        out_shape=[jax.ShapeDtypeStruct((N, H), jnp.float32),
                   jax.ShapeDtypeStruct((N, H), jnp.float32),
                   jax.ShapeDtypeStruct((N, H), jnp.float32)],
    )(msum, feat, W1, b1, W2, b2, Ws, Wd)


def _readout_body(feat_ref, W1_ref, b1_ref, W2_ref, b2_ref, sum_ref, max_ref):
    x = _silu(jnp.dot(feat_ref[...], W1_ref[...],
                      preferred_element_type=jnp.float32) + b1_ref[...])
    y = jnp.dot(x, W2_ref[...], preferred_element_type=jnp.float32) + b2_ref[...]
    psum = jnp.sum(y, axis=0, keepdims=True)
    pmax = jnp.max(y, axis=0, keepdims=True)

    @pl.when(pl.program_id(0) == 0)
    def _init():
        sum_ref[...] = psum
        max_ref[...] = pmax

    @pl.when(pl.program_id(0) != 0)
    def _acc():
        sum_ref[...] = sum_ref[...] + psum
        max_ref[...] = jnp.maximum(max_ref[...], pmax)


def _readout(feat, W1, b1, W2, b2):
    grid = N // BN
    nb = lambda i: (i, 0)
    wb = lambda i: (0, 0)
    return pl.pallas_call(
        _readout_body,
        grid=(grid,),
        in_specs=[
            pl.BlockSpec((BN, H), nb),
            pl.BlockSpec((H, H), wb),
            pl.BlockSpec((1, H), wb),
            pl.BlockSpec((H, H), wb),
            pl.BlockSpec((1, H), wb),
        ],
        out_specs=[pl.BlockSpec((1, H), wb), pl.BlockSpec((1, H), wb)],
        out_shape=[jax.ShapeDtypeStruct((1, H), jnp.float32),
                   jax.ShapeDtypeStruct((1, H), jnp.float32)],
    )(feat, W1, b1, W2, b2)


def _edge_embed_body(dist_ref, eW_ref, eb_ref, out_ref):
    out_ref[...] = _silu(_silu(dist_ref[...] * eW_ref[...] + eb_ref[...]))


def _edge_embed(edge_dist, eW, eb):
    grid = E // BE
    return pl.pallas_call(
        _edge_embed_body,
        grid=(grid,),
        in_specs=[
            pl.BlockSpec((BE, 1), lambda i: (i, 0)),
            pl.BlockSpec((1, H), lambda i: (0, 0)),
            pl.BlockSpec((1, H), lambda i: (0, 0)),
        ],
        out_specs=pl.BlockSpec((BE, H), lambda i: (i, 0)),
        out_shape=jax.ShapeDtypeStruct((E, H), jnp.float32),
    )(edge_dist, eW, eb)


def _head_body(s_ref, mx_ref, W1_ref, b1_ref, W2_ref, b2_ref, out_ref):
    s = s_ref[...]
    ro = jnp.concatenate([s, s * (1.0 / N), mx_ref[...]], axis=-1)
    h = jax.nn.relu(jnp.dot(ro, W1_ref[...],
                            preferred_element_type=jnp.float32) + b1_ref[...])
    out_ref[...] = jnp.dot(h, W2_ref[...],
                           preferred_element_type=jnp.float32) + b2_ref[...]


def _head(s, mx, W1, b1, W2, b2):
    return pl.pallas_call(
        _head_body,
        out_shape=jax.ShapeDtypeStruct((1, T), jnp.float32),
    )(s, mx, W1, b1, W2, b2)


def kernel(edge_index, edge_dist, node_emb, eW, eb, mW1, mb1, mW2, mb2, sW,
           sb, uW1, ub1, uW2, ub2, oW1, ob1, oW2, ob2, rW1, rb1, rW2, rb2):
    src = edge_index[0]
    dst = edge_index[1]
    feat = jnp.broadcast_to(node_emb[None, :], (N, H))

    de = _edge_embed(edge_dist, eW, eb[None, :])

    for i in range(DEPTH):
        Ws = mW1[i, 0:H]
        Wd = mW1[i, H:2 * H]
        We = mW1[i, 2 * H:3 * H]
        if i == 0:
            pq = node_emb[None, :] @ (Ws + Wd)
            g = jnp.broadcast_to(pq, (E, H))
        else:
            g = P[src] + Q[dst]
        m, de = _edge_mlp(g, de, We, mb1[i][None, :], mW2[i], mb2[i][None, :],
                          sW[i], sb[i][None, :])
        msum = jax.ops.segment_sum(m, dst, num_segments=N)
        if i + 1 < DEPTH:
            nWs = mW1[i + 1, 0:H]
            nWd = mW1[i + 1, H:2 * H]
        else:
            nWs = mW1[0, 0:H]
            nWd = mW1[0, H:2 * H]
        feat, P, Q = _node_update(msum, feat, uW1[i], ub1[i][None, :], uW2[i],
                                  ub2[i][None, :], nWs, nWd)

    s, mx = _readout(feat, oW1, ob1[None, :], oW2, ob2[None, :])
    out = _head(s, mx, rW1, rb1[None, :], rW2, rb2[None, :])
    return out
